# baseline (speedup 1.0000x reference)
import functools

import jax
import jax.numpy as jnp
from jax import lax
from jax.experimental import pallas as pl
from jax.experimental.pallas import tpu as pltpu
from jax.experimental.pallas import tpu_sc as plsc

ZCH_SIZE = 1_000_000
EMBED_DIM = 64
N_IDS = 327_680
CHUNK = 128
ROWS = N_IDS // CHUNK
NW = 32
RPW = ROWS // NW


def _remap_chunk(ids_v, rem_v):
    for j in range(CHUNK // 16):
        x = ids_v[pl.ds(j * 16, 16)]
        x = x - jnp.where(x >= 2 * ZCH_SIZE, jnp.int32(2 * ZCH_SIZE), jnp.int32(0))
        x = x - jnp.where(x >= ZCH_SIZE, jnp.int32(ZCH_SIZE), jnp.int32(0))
        rem_v[pl.ds(j * 16, 16)] = x


def _sc_body(ids0, ids1, t0, t1, rem0, rem1, part, ids_v, rem_v, rows_v, acc_v, sem):
    cid = lax.axis_index("c")
    sid = lax.axis_index("s")
    wid = sid * 2 + cid

    def do_feature(ids_hbm, t_hbm, rem_hbm, acc):
        def chunk_body(i, acc):
            row = wid * RPW + i
            pltpu.sync_copy(ids_hbm.at[row], ids_v)
            _remap_chunk(ids_v, rem_v)
            pltpu.sync_copy(rem_v, rem_hbm.at[row])
            pltpu.async_copy(t_hbm.at[rem_v], rows_v, sem).wait()

            def row_body(r, acc):
                a0, a1, a2, a3 = acc
                a0 = a0 + rows_v[r, pl.ds(0, 16)]
                a1 = a1 + rows_v[r, pl.ds(16, 16)]
                a2 = a2 + rows_v[r, pl.ds(32, 16)]
                a3 = a3 + rows_v[r, pl.ds(48, 16)]
                return (a0, a1, a2, a3)

            return lax.fori_loop(0, CHUNK, row_body, acc)

        return lax.fori_loop(0, RPW, chunk_body, acc)

    z = jnp.zeros((16,), jnp.float32)
    acc = (z, z, z, z)
    acc = do_feature(ids0, t0, rem0, acc)
    acc = do_feature(ids1, t1, rem1, acc)
    acc_v[...] = acc[0] + acc[1] + acc[2] + acc[3]
    pltpu.sync_copy(acc_v, part.at[wid])


@jax.jit
def kernel(ids_0, ids_1, table_0, table_1):
    mesh = plsc.VectorSubcoreMesh(core_axis_name="c", subcore_axis_name="s")
    run = functools.partial(
        pl.kernel,
        out_type=(
            jax.ShapeDtypeStruct((ROWS, CHUNK), jnp.int32),
            jax.ShapeDtypeStruct((ROWS, CHUNK), jnp.int32),
            jax.ShapeDtypeStruct((NW, 16), jnp.float32),
        ),
        mesh=mesh,
        scratch_types=(
            pltpu.VMEM((CHUNK,), jnp.int32),
            pltpu.VMEM((CHUNK,), jnp.int32),
            pltpu.VMEM((CHUNK, EMBED_DIM), jnp.float32),
            pltpu.VMEM((16,), jnp.float32),
            pltpu.SemaphoreType.DMA,
        ),
        compiler_params=pltpu.CompilerParams(use_tc_tiling_on_sc=False),
    )(_sc_body)
    rem0, rem1, part = run(
        ids_0.reshape(ROWS, CHUNK),
        ids_1.reshape(ROWS, CHUNK),
        table_0,
        table_1,
    )
    loss = part.sum() / jnp.float32(2 * N_IDS * EMBED_DIM)
    return (loss, rem0.reshape(-1), rem1.reshape(-1))


# speedup vs baseline: 1.2042x; 1.2042x over previous
import functools

import jax
import jax.numpy as jnp
from jax import lax
from jax.experimental import pallas as pl
from jax.experimental.pallas import tpu as pltpu
from jax.experimental.pallas import tpu_sc as plsc

ZCH_SIZE = 1_000_000
EMBED_DIM = 64
N_IDS = 327_680
CHUNK = 128
ROWS = N_IDS // CHUNK
NW = 32
RPW = ROWS // NW
NBUF = 4
GROUPS = RPW // NBUF


def _sc_body(ids0, ids1, t0, t1, rem0, rem1, part,
             idx_v, rows_v, acc_v, s0, s1, s2, s3):
    cid = lax.axis_index("c")
    sid = lax.axis_index("s")
    wid = sid * 2 + cid
    sems = (s0, s1, s2, s3)

    def fire(t_hbm, i, b):
        pltpu.async_copy(t_hbm.at[idx_v.at[i]], rows_v.at[b], sems[b])

    def drain(t_hbm, b):
        pltpu.make_async_copy(t_hbm.at[pl.ds(0, CHUNK)], rows_v.at[b], sems[b]).wait()

    def do_feature(ids_hbm, t_hbm, rem_hbm, acc):
        base = wid * RPW
        pltpu.sync_copy(ids_hbm.at[pl.ds(base, RPW)], idx_v)

        def remap_body(i, c):
            for j in range(CHUNK // 16):
                x = idx_v[i, pl.ds(j * 16, 16)]
                x = x - jnp.where(x >= 2 * ZCH_SIZE,
                                  jnp.int32(2 * ZCH_SIZE), jnp.int32(0))
                x = x - jnp.where(x >= ZCH_SIZE,
                                  jnp.int32(ZCH_SIZE), jnp.int32(0))
                idx_v[i, pl.ds(j * 16, 16)] = x
            return c

        lax.fori_loop(0, RPW, remap_body, 0)
        pltpu.sync_copy(idx_v, rem_hbm.at[pl.ds(base, RPW)])

        for b in range(NBUF):
            fire(t_hbm, jnp.int32(b), b)

        def group_body(g, acc):
            for b in range(NBUF):
                drain(t_hbm, b)

                def row_body(r, acc):
                    a = list(acc)
                    for u in range(4):
                        rr = r * 4 + u
                        o = (u % 2) * 4
                        a[o + 0] = a[o + 0] + rows_v[b, rr, pl.ds(0, 16)]
                        a[o + 1] = a[o + 1] + rows_v[b, rr, pl.ds(16, 16)]
                        a[o + 2] = a[o + 2] + rows_v[b, rr, pl.ds(32, 16)]
                        a[o + 3] = a[o + 3] + rows_v[b, rr, pl.ds(48, 16)]
                    return tuple(a)

                acc = lax.fori_loop(0, CHUNK // 4, row_body, acc)

                @pl.when(g < GROUPS - 1)
                def _():
                    fire(t_hbm, (g + 1) * NBUF + b, b)

            return acc

        return lax.fori_loop(0, GROUPS, group_body, acc)

    z = jnp.zeros((16,), jnp.float32)
    acc = (z,) * 8
    acc = do_feature(ids0, t0, rem0, acc)
    acc = do_feature(ids1, t1, rem1, acc)
    tot = acc[0]
    for k in range(1, 8):
        tot = tot + acc[k]
    acc_v[...] = tot
    pltpu.sync_copy(acc_v, part.at[wid])


@jax.jit
def kernel(ids_0, ids_1, table_0, table_1):
    mesh = plsc.VectorSubcoreMesh(core_axis_name="c", subcore_axis_name="s")
    run = functools.partial(
        pl.kernel,
        out_type=(
            jax.ShapeDtypeStruct((ROWS, CHUNK), jnp.int32),
            jax.ShapeDtypeStruct((ROWS, CHUNK), jnp.int32),
            jax.ShapeDtypeStruct((NW, 16), jnp.float32),
        ),
        mesh=mesh,
        scratch_types=(
            pltpu.VMEM((RPW, CHUNK), jnp.int32),
            pltpu.VMEM((NBUF, CHUNK, EMBED_DIM), jnp.float32),
            pltpu.VMEM((16,), jnp.float32),
            pltpu.SemaphoreType.DMA,
            pltpu.SemaphoreType.DMA,
            pltpu.SemaphoreType.DMA,
            pltpu.SemaphoreType.DMA,
        ),
        compiler_params=pltpu.CompilerParams(use_tc_tiling_on_sc=False),
    )(_sc_body)
    rem0, rem1, part = run(
        ids_0.reshape(ROWS, CHUNK),
        ids_1.reshape(ROWS, CHUNK),
        table_0,
        table_1,
    )
    loss = part.sum() / jnp.float32(2 * N_IDS * EMBED_DIM)
    return (loss, rem0.reshape(-1), rem1.reshape(-1))
